# Initial kernel scaffold; baseline (speedup 1.0000x reference)
import jax
import jax.numpy as jnp
from jax.experimental import pallas as pl


def kernel(x, edge_index, W1, b1, W2, b2, Wc, bc):
    raise NotImplementedError("write your pallas kernel here")


# speedup vs baseline: 27.9907x; 27.9907x over previous
import functools

import jax
import jax.numpy as jnp
from jax import lax
from jax.experimental import pallas as pl
from jax.experimental.pallas import tpu as pltpu
from jax.experimental.pallas import tpu_sc as plsc

_NC = 2
_NS = 16
_CH = 128
_DD = 16


def _sc_mesh():
    return plsc.VectorSubcoreMesh(
        core_axis_name="c", subcore_axis_name="s",
        num_cores=_NC, num_subcores=_NS)


def _make_deg_kernel(n_pad, k):
    rows = n_pad // _NS

    def body(dst_hbm, ones_hbm, zeros_hbm, out_hbm, dst_v, ones_v, acc):
        c = lax.axis_index("c")
        s = lax.axis_index("s")
        r0 = s * rows
        pltpu.sync_copy(zeros_hbm.at[pl.ds(r0, rows)], acc.at[pl.ds(r0, rows)])
        pltpu.sync_copy(dst_hbm.at[c, s], dst_v)
        pltpu.sync_copy(ones_hbm, ones_v)
        plsc.subcore_barrier()

        def step(j, carry):
            pltpu.sync_copy(ones_v, acc.at[dst_v.at[j]], add=True)
            return carry

        lax.fori_loop(0, k, step, 0)
        plsc.subcore_barrier()
        pltpu.sync_copy(acc.at[pl.ds(r0, rows)],
                        out_hbm.at[c, pl.ds(r0, rows)])

    return pl.kernel(
        body,
        out_type=jax.ShapeDtypeStruct((_NC, n_pad, _DD), jnp.float32),
        mesh=_sc_mesh(),
        compiler_params=pltpu.CompilerParams(use_tc_tiling_on_sc=False),
        scratch_types=[
            pltpu.VMEM((k, _CH), jnp.int32),
            pltpu.VMEM((_CH, _DD), jnp.float32),
            pltpu.VMEM_SHARED((n_pad, _DD), jnp.float32),
        ],
    )


def _make_agg_kernel(n_pad, k, d_feat):
    rows = n_pad // _NS

    def body(g_hbm, src_hbm, dst_hbm, zeros_hbm, out_hbm,
             src_v, dst_v, rows_v, sem, acc):
        c = lax.axis_index("c")
        s = lax.axis_index("s")
        r0 = s * rows
        pltpu.sync_copy(zeros_hbm.at[pl.ds(r0, rows)], acc.at[pl.ds(r0, rows)])
        pltpu.sync_copy(src_hbm.at[c, s], src_v)
        pltpu.sync_copy(dst_hbm.at[c, s], dst_v)
        plsc.subcore_barrier()

        def step(j, carry):
            pltpu.async_copy(g_hbm.at[src_v.at[j]], rows_v, sem).wait()
            pltpu.sync_copy(rows_v, acc.at[dst_v.at[j]], add=True)
            return carry

        lax.fori_loop(0, k, step, 0)
        plsc.subcore_barrier()
        pltpu.sync_copy(acc.at[pl.ds(r0, rows)],
                        out_hbm.at[c, pl.ds(r0, rows)])

    return pl.kernel(
        body,
        out_type=jax.ShapeDtypeStruct((_NC, n_pad, d_feat), jnp.float32),
        mesh=_sc_mesh(),
        compiler_params=pltpu.CompilerParams(use_tc_tiling_on_sc=False),
        scratch_types=[
            pltpu.VMEM((k, _CH), jnp.int32),
            pltpu.VMEM((k, _CH), jnp.int32),
            pltpu.VMEM((_CH, d_feat), jnp.float32),
            pltpu.SemaphoreType.DMA,
            pltpu.VMEM_SHARED((n_pad, d_feat), jnp.float32),
        ],
    )


def _tc_a_body(dp_ref, x_ref, w1_ref, g1_ref, d_ref):
    deg = dp_ref[0, :, 0:1] + dp_ref[1, :, 0:1] + 1.0
    dd = lax.rsqrt(deg)
    d_ref[...] = dd
    h1 = jnp.dot(x_ref[...], w1_ref[...], preferred_element_type=jnp.float32)
    g1_ref[...] = dd * h1


def _tc_b_body(ap_ref, g1_ref, d_ref, b1_ref, w2_ref, g2_ref):
    t = ap_ref[0] + ap_ref[1] + g1_ref[...]
    z1 = jnp.maximum(d_ref[...] * t + b1_ref[...], 0.0)
    h2 = jnp.dot(z1, w2_ref[...], preferred_element_type=jnp.float32)
    g2_ref[...] = d_ref[...] * h2


def _tc_c_body(ap_ref, g2_ref, d_ref, b2_ref, wc_ref, bc_ref, out_ref):
    t = ap_ref[0] + ap_ref[1] + g2_ref[...]
    z2 = jnp.maximum(d_ref[...] * t + b2_ref[...], 0.0)
    logits = jnp.dot(z2, wc_ref[...],
                     preferred_element_type=jnp.float32) + bc_ref[...]
    m = jnp.max(logits, axis=1, keepdims=True)
    lse = m + jnp.log(jnp.sum(jnp.exp(logits - m), axis=1, keepdims=True))
    out_ref[...] = logits - lse


def kernel(x, edge_index, W1, b1, W2, b2, Wc, bc):
    n, f = x.shape
    e = edge_index.shape[1]
    d1 = W1.shape[1]
    d2 = W2.shape[1]
    ncls = Wc.shape[1]

    tiles = _NC * _NS
    k = -(-e // (tiles * _CH))
    e_pad = tiles * k * _CH
    n_pad = -(-(n + 1) // (8 * _NS)) * (8 * _NS)

    ei = edge_index.astype(jnp.int32)
    pad = jnp.full((e_pad - e,), n, jnp.int32)
    src = jnp.concatenate([ei[0], pad]).reshape(_NC, _NS, k, _CH)
    dst = jnp.concatenate([ei[1], pad]).reshape(_NC, _NS, k, _CH)
    x_p = jnp.concatenate([x, jnp.zeros((n_pad - n, f), x.dtype)])

    ones_dd = jnp.ones((_CH, _DD), jnp.float32)
    z_dd = jnp.zeros((n_pad, _DD), jnp.float32)
    z_d1 = jnp.zeros((n_pad, d1), jnp.float32)
    z_d2 = jnp.zeros((n_pad, d2), jnp.float32)

    deg_p = _make_deg_kernel(n_pad, k)(dst, ones_dd, z_dd)

    rb = 1264
    grid = (n_pad // rb,)

    g1, d = pl.pallas_call(
        _tc_a_body,
        grid=grid,
        in_specs=[
            pl.BlockSpec((_NC, rb, _DD), lambda i: (0, i, 0)),
            pl.BlockSpec((rb, f), lambda i: (i, 0)),
            pl.BlockSpec((f, d1), lambda i: (0, 0)),
        ],
        out_specs=[
            pl.BlockSpec((rb, d1), lambda i: (i, 0)),
            pl.BlockSpec((rb, 1), lambda i: (i, 0)),
        ],
        out_shape=[
            jax.ShapeDtypeStruct((n_pad, d1), jnp.float32),
            jax.ShapeDtypeStruct((n_pad, 1), jnp.float32),
        ],
    )(deg_p, x_p, W1)

    agg1 = _make_agg_kernel(n_pad, k, d1)(g1, src, dst, z_d1)

    g2 = pl.pallas_call(
        _tc_b_body,
        grid=grid,
        in_specs=[
            pl.BlockSpec((_NC, rb, d1), lambda i: (0, i, 0)),
            pl.BlockSpec((rb, d1), lambda i: (i, 0)),
            pl.BlockSpec((rb, 1), lambda i: (i, 0)),
            pl.BlockSpec((1, d1), lambda i: (0, 0)),
            pl.BlockSpec((d1, d2), lambda i: (0, 0)),
        ],
        out_specs=pl.BlockSpec((rb, d2), lambda i: (i, 0)),
        out_shape=jax.ShapeDtypeStruct((n_pad, d2), jnp.float32),
    )(agg1, g1, d, b1.reshape(1, d1), W2)

    agg2 = _make_agg_kernel(n_pad, k, d2)(g2, src, dst, z_d2)

    out = pl.pallas_call(
        _tc_c_body,
        grid=grid,
        in_specs=[
            pl.BlockSpec((_NC, rb, d2), lambda i: (0, i, 0)),
            pl.BlockSpec((rb, d2), lambda i: (i, 0)),
            pl.BlockSpec((rb, 1), lambda i: (i, 0)),
            pl.BlockSpec((1, d2), lambda i: (0, 0)),
            pl.BlockSpec((d2, ncls), lambda i: (0, 0)),
            pl.BlockSpec((1, ncls), lambda i: (0, 0)),
        ],
        out_specs=pl.BlockSpec((rb, ncls), lambda i: (i, 0)),
        out_shape=jax.ShapeDtypeStruct((n_pad, ncls), jnp.float32),
    )(agg2, g2, d, b2.reshape(1, d2), Wc, bc.reshape(1, ncls))

    return out[:n]


# speedup vs baseline: 37.1007x; 1.3255x over previous
import functools

import jax
import jax.numpy as jnp
from jax import lax
from jax.experimental import pallas as pl
from jax.experimental.pallas import tpu as pltpu
from jax.experimental.pallas import tpu_sc as plsc

_NC = 2
_NS = 16
_CH = 128
_DD = 16


def _sc_mesh():
    return plsc.VectorSubcoreMesh(
        core_axis_name="c", subcore_axis_name="s",
        num_cores=_NC, num_subcores=_NS)


def _make_deg_kernel(n_pad, k):
    rows = n_pad // _NS

    def body(dst_hbm, ones_hbm, zeros_hbm, out_hbm, dst_v, ones_v, ssems, acc):
        c = lax.axis_index("c")
        s = lax.axis_index("s")
        r0 = s * rows
        pltpu.sync_copy(zeros_hbm.at[pl.ds(r0, rows)], acc.at[pl.ds(r0, rows)])
        pltpu.sync_copy(dst_hbm.at[c, s], dst_v)
        pltpu.sync_copy(ones_hbm, ones_v)
        plsc.subcore_barrier()

        def step(j, carry):
            @pl.when(j >= 2)
            def _():
                jm = j - 2
                pltpu.make_async_copy(
                    ones_v, acc.at[dst_v.at[jm]], ssems.at[lax.rem(jm, 2)]
                ).wait()
            pltpu.async_copy(ones_v, acc.at[dst_v.at[j]],
                             ssems.at[lax.rem(j, 2)], add=True)
            return carry

        lax.fori_loop(0, k, step, 0)
        for dj in (k - 2, k - 1):
            pltpu.make_async_copy(
                ones_v, acc.at[dst_v.at[dj]], ssems.at[dj % 2]).wait()
        plsc.subcore_barrier()
        pltpu.sync_copy(acc.at[pl.ds(r0, rows)],
                        out_hbm.at[c, pl.ds(r0, rows)])

    return pl.kernel(
        body,
        out_type=jax.ShapeDtypeStruct((_NC, n_pad, _DD), jnp.float32),
        mesh=_sc_mesh(),
        compiler_params=pltpu.CompilerParams(use_tc_tiling_on_sc=False),
        scratch_types=[
            pltpu.VMEM((k, _CH), jnp.int32),
            pltpu.VMEM((_CH, _DD), jnp.float32),
            pltpu.SemaphoreType.DMA((2,)),
            pltpu.VMEM_SHARED((n_pad, _DD), jnp.float32),
        ],
    )


def _make_agg_kernel(n_pad, k, d_feat):
    rows = n_pad // _NS

    def body(g_hbm, src_hbm, dst_hbm, zeros_hbm, out_hbm,
             src_v, dst_v, rows_v, gsems, ssems, acc):
        c = lax.axis_index("c")
        s = lax.axis_index("s")
        r0 = s * rows
        pltpu.sync_copy(zeros_hbm.at[pl.ds(r0, rows)], acc.at[pl.ds(r0, rows)])
        pltpu.sync_copy(src_hbm.at[c, s], src_v)
        pltpu.sync_copy(dst_hbm.at[c, s], dst_v)
        plsc.subcore_barrier()

        pltpu.async_copy(g_hbm.at[src_v.at[0]], rows_v.at[0], gsems.at[0])
        pltpu.async_copy(g_hbm.at[src_v.at[1]], rows_v.at[1], gsems.at[1])

        def step(j, carry):
            b4 = lax.rem(j, 4)
            pltpu.make_async_copy(
                g_hbm.at[src_v.at[j]], rows_v.at[b4], gsems.at[b4]).wait()

            @pl.when(j >= 2)
            def _():
                jm = j - 2
                pltpu.make_async_copy(
                    rows_v.at[lax.rem(jm, 4)], acc.at[dst_v.at[jm]],
                    ssems.at[lax.rem(jm, 2)]).wait()

            @pl.when(j + 2 < k)
            def _():
                jp = j + 2
                bp = lax.rem(jp, 4)
                pltpu.async_copy(
                    g_hbm.at[src_v.at[jp]], rows_v.at[bp], gsems.at[bp])

            pltpu.async_copy(rows_v.at[b4], acc.at[dst_v.at[j]],
                             ssems.at[lax.rem(j, 2)], add=True)
            return carry

        lax.fori_loop(0, k, step, 0)
        for dj in (k - 2, k - 1):
            pltpu.make_async_copy(
                rows_v.at[dj % 4], acc.at[dst_v.at[dj]],
                ssems.at[dj % 2]).wait()
        plsc.subcore_barrier()
        pltpu.sync_copy(acc.at[pl.ds(r0, rows)],
                        out_hbm.at[c, pl.ds(r0, rows)])

    return pl.kernel(
        body,
        out_type=jax.ShapeDtypeStruct((_NC, n_pad, d_feat), jnp.float32),
        mesh=_sc_mesh(),
        compiler_params=pltpu.CompilerParams(use_tc_tiling_on_sc=False),
        scratch_types=[
            pltpu.VMEM((k, _CH), jnp.int32),
            pltpu.VMEM((k, _CH), jnp.int32),
            pltpu.VMEM((4, _CH, d_feat), jnp.float32),
            pltpu.SemaphoreType.DMA((4,)),
            pltpu.SemaphoreType.DMA((2,)),
            pltpu.VMEM_SHARED((n_pad, d_feat), jnp.float32),
        ],
    )


def _tc_a_body(dp_ref, x_ref, w1_ref, g1_ref, d_ref):
    deg = dp_ref[0, :, 0:1] + dp_ref[1, :, 0:1] + 1.0
    dd = lax.rsqrt(deg)
    d_ref[...] = dd
    h1 = jnp.dot(x_ref[...], w1_ref[...], preferred_element_type=jnp.float32)
    g1_ref[...] = dd * h1


def _tc_b_body(ap_ref, g1_ref, d_ref, b1_ref, w2_ref, g2_ref):
    t = ap_ref[0] + ap_ref[1] + g1_ref[...]
    z1 = jnp.maximum(d_ref[...] * t + b1_ref[...], 0.0)
    h2 = jnp.dot(z1, w2_ref[...], preferred_element_type=jnp.float32)
    g2_ref[...] = d_ref[...] * h2


def _tc_c_body(ap_ref, g2_ref, d_ref, b2_ref, wc_ref, bc_ref, out_ref):
    t = ap_ref[0] + ap_ref[1] + g2_ref[...]
    z2 = jnp.maximum(d_ref[...] * t + b2_ref[...], 0.0)
    logits = jnp.dot(z2, wc_ref[...],
                     preferred_element_type=jnp.float32) + bc_ref[...]
    m = jnp.max(logits, axis=1, keepdims=True)
    lse = m + jnp.log(jnp.sum(jnp.exp(logits - m), axis=1, keepdims=True))
    out_ref[...] = logits - lse


def kernel(x, edge_index, W1, b1, W2, b2, Wc, bc):
    n, f = x.shape
    e = edge_index.shape[1]
    d1 = W1.shape[1]
    d2 = W2.shape[1]
    ncls = Wc.shape[1]

    tiles = _NC * _NS
    k = -(-e // (tiles * _CH))
    e_pad = tiles * k * _CH
    n_pad = -(-(n + 1) // (8 * _NS)) * (8 * _NS)

    ei = edge_index.astype(jnp.int32)
    pad = jnp.full((e_pad - e,), n, jnp.int32)
    src = jnp.concatenate([ei[0], pad]).reshape(_NC, _NS, k, _CH)
    dst = jnp.concatenate([ei[1], pad]).reshape(_NC, _NS, k, _CH)
    x_p = jnp.concatenate([x, jnp.zeros((n_pad - n, f), x.dtype)])

    ones_dd = jnp.ones((_CH, _DD), jnp.float32)
    z_dd = jnp.zeros((n_pad, _DD), jnp.float32)
    z_d1 = jnp.zeros((n_pad, d1), jnp.float32)
    z_d2 = jnp.zeros((n_pad, d2), jnp.float32)

    deg_p = _make_deg_kernel(n_pad, k)(dst, ones_dd, z_dd)

    rb = 1264
    grid = (n_pad // rb,)

    g1, d = pl.pallas_call(
        _tc_a_body,
        grid=grid,
        in_specs=[
            pl.BlockSpec((_NC, rb, _DD), lambda i: (0, i, 0)),
            pl.BlockSpec((rb, f), lambda i: (i, 0)),
            pl.BlockSpec((f, d1), lambda i: (0, 0)),
        ],
        out_specs=[
            pl.BlockSpec((rb, d1), lambda i: (i, 0)),
            pl.BlockSpec((rb, 1), lambda i: (i, 0)),
        ],
        out_shape=[
            jax.ShapeDtypeStruct((n_pad, d1), jnp.float32),
            jax.ShapeDtypeStruct((n_pad, 1), jnp.float32),
        ],
    )(deg_p, x_p, W1)

    agg1 = _make_agg_kernel(n_pad, k, d1)(g1, src, dst, z_d1)

    g2 = pl.pallas_call(
        _tc_b_body,
        grid=grid,
        in_specs=[
            pl.BlockSpec((_NC, rb, d1), lambda i: (0, i, 0)),
            pl.BlockSpec((rb, d1), lambda i: (i, 0)),
            pl.BlockSpec((rb, 1), lambda i: (i, 0)),
            pl.BlockSpec((1, d1), lambda i: (0, 0)),
            pl.BlockSpec((d1, d2), lambda i: (0, 0)),
        ],
        out_specs=pl.BlockSpec((rb, d2), lambda i: (i, 0)),
        out_shape=jax.ShapeDtypeStruct((n_pad, d2), jnp.float32),
    )(agg1, g1, d, b1.reshape(1, d1), W2)

    agg2 = _make_agg_kernel(n_pad, k, d2)(g2, src, dst, z_d2)

    out = pl.pallas_call(
        _tc_c_body,
        grid=grid,
        in_specs=[
            pl.BlockSpec((_NC, rb, d2), lambda i: (0, i, 0)),
            pl.BlockSpec((rb, d2), lambda i: (i, 0)),
            pl.BlockSpec((rb, 1), lambda i: (i, 0)),
            pl.BlockSpec((1, d2), lambda i: (0, 0)),
            pl.BlockSpec((d2, ncls), lambda i: (0, 0)),
            pl.BlockSpec((1, ncls), lambda i: (0, 0)),
        ],
        out_specs=pl.BlockSpec((rb, ncls), lambda i: (i, 0)),
        out_shape=jax.ShapeDtypeStruct((n_pad, ncls), jnp.float32),
    )(agg2, g2, d, b2.reshape(1, d2), Wc, bc.reshape(1, ncls))

    return out[:n]


# speedup vs baseline: 39.6288x; 1.0681x over previous
import functools

import jax
import jax.numpy as jnp
from jax import lax
from jax.experimental import pallas as pl
from jax.experimental.pallas import tpu as pltpu
from jax.experimental.pallas import tpu_sc as plsc

_NC = 2
_NS = 16
_CH = 128
_DD = 8


def _sc_mesh():
    return plsc.VectorSubcoreMesh(
        core_axis_name="c", subcore_axis_name="s",
        num_cores=_NC, num_subcores=_NS)


def _make_deg_kernel(n_pad, k):
    rows = n_pad // _NS

    def body(dst_hbm, ones_hbm, zeros_hbm, out_hbm, dst_v, ones_v, ssems, acc):
        c = lax.axis_index("c")
        s = lax.axis_index("s")
        r0 = s * rows
        pltpu.sync_copy(zeros_hbm.at[pl.ds(r0, rows)], acc.at[pl.ds(r0, rows)])
        pltpu.sync_copy(dst_hbm.at[c, s], dst_v)
        pltpu.sync_copy(ones_hbm, ones_v)
        plsc.subcore_barrier()

        def step(j, carry):
            @pl.when(j >= 2)
            def _():
                jm = j - 2
                pltpu.make_async_copy(
                    ones_v, acc.at[dst_v.at[jm]], ssems.at[lax.rem(jm, 2)]
                ).wait()
            pltpu.async_copy(ones_v, acc.at[dst_v.at[j]],
                             ssems.at[lax.rem(j, 2)], add=True)
            return carry

        lax.fori_loop(0, k, step, 0)
        for dj in (k - 2, k - 1):
            pltpu.make_async_copy(
                ones_v, acc.at[dst_v.at[dj]], ssems.at[dj % 2]).wait()
        plsc.subcore_barrier()
        pltpu.sync_copy(acc.at[pl.ds(r0, rows)],
                        out_hbm.at[c, pl.ds(r0, rows)])

    return pl.kernel(
        body,
        out_type=jax.ShapeDtypeStruct((_NC, n_pad, _DD), jnp.float32),
        mesh=_sc_mesh(),
        compiler_params=pltpu.CompilerParams(use_tc_tiling_on_sc=False),
        scratch_types=[
            pltpu.VMEM((k, _CH), jnp.int32),
            pltpu.VMEM((_CH, _DD), jnp.float32),
            pltpu.SemaphoreType.DMA((2,)),
            pltpu.VMEM_SHARED((n_pad, _DD), jnp.float32),
        ],
    )


def _make_agg_kernel(n_pad, k, d_feat):
    rows = n_pad // _NS

    def body(g_hbm, src_hbm, dst_hbm, zeros_hbm, out_hbm,
             src_v, dst_v, rows_v, gsems, ssems, acc):
        c = lax.axis_index("c")
        s = lax.axis_index("s")
        r0 = s * rows
        pltpu.sync_copy(zeros_hbm.at[pl.ds(r0, rows)], acc.at[pl.ds(r0, rows)])
        pltpu.sync_copy(src_hbm.at[c, s], src_v)
        pltpu.sync_copy(dst_hbm.at[c, s], dst_v)
        plsc.subcore_barrier()

        pltpu.async_copy(g_hbm.at[src_v.at[0]], rows_v.at[0], gsems.at[0])
        pltpu.async_copy(g_hbm.at[src_v.at[1]], rows_v.at[1], gsems.at[1])

        def step(j, carry):
            b4 = lax.rem(j, 4)
            pltpu.make_async_copy(
                g_hbm.at[src_v.at[j]], rows_v.at[b4], gsems.at[b4]).wait()

            @pl.when(j >= 2)
            def _():
                jm = j - 2
                pltpu.make_async_copy(
                    rows_v.at[lax.rem(jm, 4)], acc.at[dst_v.at[jm]],
                    ssems.at[lax.rem(jm, 2)]).wait()

            @pl.when(j + 2 < k)
            def _():
                jp = j + 2
                bp = lax.rem(jp, 4)
                pltpu.async_copy(
                    g_hbm.at[src_v.at[jp]], rows_v.at[bp], gsems.at[bp])

            pltpu.async_copy(rows_v.at[b4], acc.at[dst_v.at[j]],
                             ssems.at[lax.rem(j, 2)], add=True)
            return carry

        lax.fori_loop(0, k, step, 0)
        for dj in (k - 2, k - 1):
            pltpu.make_async_copy(
                rows_v.at[dj % 4], acc.at[dst_v.at[dj]],
                ssems.at[dj % 2]).wait()
        plsc.subcore_barrier()
        pltpu.sync_copy(acc.at[pl.ds(r0, rows)],
                        out_hbm.at[c, pl.ds(r0, rows)])

    return pl.kernel(
        body,
        out_type=jax.ShapeDtypeStruct((_NC, n_pad, d_feat), jnp.float32),
        mesh=_sc_mesh(),
        compiler_params=pltpu.CompilerParams(use_tc_tiling_on_sc=False),
        scratch_types=[
            pltpu.VMEM((k, _CH), jnp.int32),
            pltpu.VMEM((k, _CH), jnp.int32),
            pltpu.VMEM((4, _CH, d_feat), jnp.float32),
            pltpu.SemaphoreType.DMA((4,)),
            pltpu.SemaphoreType.DMA((2,)),
            pltpu.VMEM_SHARED((n_pad, d_feat), jnp.float32),
        ],
    )


def _tc_a1_body(x_ref, w1_ref, h1_ref):
    h1_ref[...] = jnp.dot(x_ref[...], w1_ref[...],
                          preferred_element_type=jnp.float32)


def _tc_a2_body(dp_ref, h1_ref, g1_ref, d_ref):
    deg = dp_ref[0, :, 0:1] + dp_ref[1, :, 0:1] + 1.0
    dd = lax.rsqrt(deg)
    d_ref[...] = dd
    g1_ref[...] = dd * h1_ref[...]


def _tc_b_body(ap_ref, g1_ref, d_ref, b1_ref, w2_ref, g2_ref):
    t = ap_ref[0] + ap_ref[1] + g1_ref[...]
    z1 = jnp.maximum(d_ref[...] * t + b1_ref[...], 0.0)
    h2 = jnp.dot(z1, w2_ref[...], preferred_element_type=jnp.float32)
    g2_ref[...] = d_ref[...] * h2


def _tc_c_body(ap_ref, g2_ref, d_ref, b2_ref, wc_ref, bc_ref, out_ref):
    t = ap_ref[0] + ap_ref[1] + g2_ref[...]
    z2 = jnp.maximum(d_ref[...] * t + b2_ref[...], 0.0)
    logits = jnp.dot(z2, wc_ref[...],
                     preferred_element_type=jnp.float32) + bc_ref[...]
    m = jnp.max(logits, axis=1, keepdims=True)
    lse = m + jnp.log(jnp.sum(jnp.exp(logits - m), axis=1, keepdims=True))
    out_ref[...] = logits - lse


def kernel(x, edge_index, W1, b1, W2, b2, Wc, bc):
    n, f = x.shape
    e = edge_index.shape[1]
    d1 = W1.shape[1]
    d2 = W2.shape[1]
    ncls = Wc.shape[1]

    tiles = _NC * _NS
    k = -(-e // (tiles * _CH))
    e_pad = tiles * k * _CH
    n_pad = -(-(n + 1) // (8 * _NS)) * (8 * _NS)

    ei = edge_index.astype(jnp.int32)
    pad = jnp.full((e_pad - e,), n, jnp.int32)
    src = jnp.concatenate([ei[0], pad]).reshape(_NC, _NS, k, _CH)
    dst = jnp.concatenate([ei[1], pad]).reshape(_NC, _NS, k, _CH)

    ones_dd = jnp.ones((_CH, _DD), jnp.float32)
    z_dd = jnp.zeros((n_pad, _DD), jnp.float32)
    z_d1 = jnp.zeros((n_pad, d1), jnp.float32)
    z_d2 = jnp.zeros((n_pad, d2), jnp.float32)

    rb = 1000
    grid = (n // rb,)

    h1 = pl.pallas_call(
        _tc_a1_body,
        grid=grid,
        in_specs=[
            pl.BlockSpec((rb, f), lambda i: (i, 0)),
            pl.BlockSpec((f, d1), lambda i: (0, 0)),
        ],
        out_specs=pl.BlockSpec((rb, d1), lambda i: (i, 0)),
        out_shape=jax.ShapeDtypeStruct((n_pad, d1), jnp.float32),
    )(x, W1)

    deg_p = _make_deg_kernel(n_pad, k)(dst, ones_dd, z_dd)

    g1, d = pl.pallas_call(
        _tc_a2_body,
        grid=grid,
        in_specs=[
            pl.BlockSpec((_NC, rb, _DD), lambda i: (0, i, 0)),
            pl.BlockSpec((rb, d1), lambda i: (i, 0)),
        ],
        out_specs=[
            pl.BlockSpec((rb, d1), lambda i: (i, 0)),
            pl.BlockSpec((rb, 1), lambda i: (i, 0)),
        ],
        out_shape=[
            jax.ShapeDtypeStruct((n_pad, d1), jnp.float32),
            jax.ShapeDtypeStruct((n, 1), jnp.float32),
        ],
    )(deg_p, h1)

    agg1 = _make_agg_kernel(n_pad, k, d1)(g1, src, dst, z_d1)

    g2 = pl.pallas_call(
        _tc_b_body,
        grid=grid,
        in_specs=[
            pl.BlockSpec((_NC, rb, d1), lambda i: (0, i, 0)),
            pl.BlockSpec((rb, d1), lambda i: (i, 0)),
            pl.BlockSpec((rb, 1), lambda i: (i, 0)),
            pl.BlockSpec((1, d1), lambda i: (0, 0)),
            pl.BlockSpec((d1, d2), lambda i: (0, 0)),
        ],
        out_specs=pl.BlockSpec((rb, d2), lambda i: (i, 0)),
        out_shape=jax.ShapeDtypeStruct((n_pad, d2), jnp.float32),
    )(agg1, g1, d, b1.reshape(1, d1), W2)

    agg2 = _make_agg_kernel(n_pad, k, d2)(g2, src, dst, z_d2)

    out = pl.pallas_call(
        _tc_c_body,
        grid=grid,
        in_specs=[
            pl.BlockSpec((_NC, rb, d2), lambda i: (0, i, 0)),
            pl.BlockSpec((rb, d2), lambda i: (i, 0)),
            pl.BlockSpec((rb, 1), lambda i: (i, 0)),
            pl.BlockSpec((1, d2), lambda i: (0, 0)),
            pl.BlockSpec((d2, ncls), lambda i: (0, 0)),
            pl.BlockSpec((1, ncls), lambda i: (0, 0)),
        ],
        out_specs=pl.BlockSpec((rb, ncls), lambda i: (i, 0)),
        out_shape=jax.ShapeDtypeStruct((n, ncls), jnp.float32),
    )(agg2, g2, d, b2.reshape(1, d2), Wc, bc.reshape(1, ncls))

    return out


# speedup vs baseline: 56.2952x; 1.4206x over previous
import jax
import jax.numpy as jnp
from jax import lax
from jax.experimental import pallas as pl
from jax.experimental.pallas import tpu as pltpu
from jax.experimental.pallas import tpu_sc as plsc

_NC = 2
_NS = 16
_CH = 128
_DD = 32


def _sc_mesh():
    return plsc.VectorSubcoreMesh(
        core_axis_name="c", subcore_axis_name="s",
        num_cores=_NC, num_subcores=_NS)


def _tile_chunks(n_chunks):
    base = n_chunks // (_NC * _NS)
    extra = n_chunks - base * _NC * _NS

    def start_cnt(c, s):
        t = c * _NS + s
        start = base * t + jnp.minimum(t, extra)
        cnt = base + jnp.where(t < extra, 1, 0)
        return start, cnt

    return base, extra, start_cnt


def _make_deg_kernel(n_pad, n_chunks):
    rows = n_pad // _NS
    base, extra, start_cnt = _tile_chunks(n_chunks)
    kmax = base + (1 if extra else 0)

    def body(dst_hbm, ones_hbm, zeros_hbm, out_hbm, dst_v, ones_v, ssems, acc):
        c = lax.axis_index("c")
        s = lax.axis_index("s")
        start, cnt = start_cnt(c, s)
        r0 = s * rows
        pltpu.sync_copy(zeros_hbm.at[pl.ds(r0, rows)], acc.at[pl.ds(r0, rows)])

        @pl.when(cnt == kmax)
        def _():
            pltpu.sync_copy(dst_hbm.at[pl.ds(start, kmax)], dst_v)

        @pl.when(cnt == base)
        def _():
            pltpu.sync_copy(dst_hbm.at[pl.ds(start, base)],
                            dst_v.at[pl.ds(0, base)])

        pltpu.sync_copy(ones_hbm, ones_v)
        plsc.subcore_barrier()

        def step(j, carry):
            @pl.when(j >= 2)
            def _():
                jm = j - 2
                pltpu.make_async_copy(
                    ones_v, acc.at[dst_v.at[jm]], ssems.at[lax.rem(jm, 2)]
                ).wait()
            pltpu.async_copy(ones_v, acc.at[dst_v.at[j]],
                             ssems.at[lax.rem(j, 2)], add=True)
            return carry

        lax.fori_loop(0, cnt, step, 0)
        for dj in (cnt - 2, cnt - 1):
            pltpu.make_async_copy(
                ones_v, acc.at[dst_v.at[dj]], ssems.at[lax.rem(dj, 2)]).wait()
        plsc.subcore_barrier()
        pltpu.sync_copy(acc.at[pl.ds(r0, rows)],
                        out_hbm.at[c, pl.ds(r0, rows)])

    return pl.kernel(
        body,
        out_type=jax.ShapeDtypeStruct((_NC, n_pad, _DD), jnp.float32),
        mesh=_sc_mesh(),
        compiler_params=pltpu.CompilerParams(use_tc_tiling_on_sc=False),
        scratch_types=[
            pltpu.VMEM((kmax, _CH), jnp.int32),
            pltpu.VMEM((_CH, _DD), jnp.float32),
            pltpu.SemaphoreType.DMA((2,)),
            pltpu.VMEM_SHARED((n_pad, _DD), jnp.float32),
        ],
    )


def _make_agg_kernel(n_pad, n_chunks, d_feat):
    rows = n_pad // _NS
    base, extra, start_cnt = _tile_chunks(n_chunks)
    kmax = base + (1 if extra else 0)

    def body(g_hbm, src_hbm, dst_hbm, zeros_hbm, out_hbm,
             src_v, dst_v, rows_v, gsems, ssems, acc):
        c = lax.axis_index("c")
        s = lax.axis_index("s")
        start, cnt = start_cnt(c, s)
        r0 = s * rows
        pltpu.sync_copy(zeros_hbm.at[pl.ds(r0, rows)], acc.at[pl.ds(r0, rows)])

        @pl.when(cnt == kmax)
        def _():
            pltpu.sync_copy(src_hbm.at[pl.ds(start, kmax)], src_v)
            pltpu.sync_copy(dst_hbm.at[pl.ds(start, kmax)], dst_v)

        @pl.when(cnt == base)
        def _():
            pltpu.sync_copy(src_hbm.at[pl.ds(start, base)],
                            src_v.at[pl.ds(0, base)])
            pltpu.sync_copy(dst_hbm.at[pl.ds(start, base)],
                            dst_v.at[pl.ds(0, base)])

        plsc.subcore_barrier()

        pltpu.async_copy(g_hbm.at[src_v.at[0]], rows_v.at[0], gsems.at[0])
        pltpu.async_copy(g_hbm.at[src_v.at[1]], rows_v.at[1], gsems.at[1])

        def step(j, carry):
            b4 = lax.rem(j, 4)
            pltpu.make_async_copy(
                g_hbm.at[src_v.at[j]], rows_v.at[b4], gsems.at[b4]).wait()

            @pl.when(j >= 2)
            def _():
                jm = j - 2
                pltpu.make_async_copy(
                    rows_v.at[lax.rem(jm, 4)], acc.at[dst_v.at[jm]],
                    ssems.at[lax.rem(jm, 2)]).wait()

            @pl.when(j + 2 < cnt)
            def _():
                jp = j + 2
                bp = lax.rem(jp, 4)
                pltpu.async_copy(
                    g_hbm.at[src_v.at[jp]], rows_v.at[bp], gsems.at[bp])

            pltpu.async_copy(rows_v.at[b4], acc.at[dst_v.at[j]],
                             ssems.at[lax.rem(j, 2)], add=True)
            return carry

        lax.fori_loop(0, cnt, step, 0)
        for dj in (cnt - 2, cnt - 1):
            pltpu.make_async_copy(
                rows_v.at[lax.rem(dj, 4)], acc.at[dst_v.at[dj]],
                ssems.at[lax.rem(dj, 2)]).wait()
        plsc.subcore_barrier()
        pltpu.sync_copy(acc.at[pl.ds(r0, rows)],
                        out_hbm.at[c, pl.ds(r0, rows)])

    return pl.kernel(
        body,
        out_type=jax.ShapeDtypeStruct((_NC, n_pad, d_feat), jnp.float32),
        mesh=_sc_mesh(),
        compiler_params=pltpu.CompilerParams(use_tc_tiling_on_sc=False),
        scratch_types=[
            pltpu.VMEM((kmax, _CH), jnp.int32),
            pltpu.VMEM((kmax, _CH), jnp.int32),
            pltpu.VMEM((4, _CH, d_feat), jnp.float32),
            pltpu.SemaphoreType.DMA((4,)),
            pltpu.SemaphoreType.DMA((2,)),
            pltpu.VMEM_SHARED((n_pad, d_feat), jnp.float32),
        ],
    )


def _tc_a1_body(xv_ref, w1bd_ref, h1_ref):
    h1_ref[...] = jnp.dot(xv_ref[...], w1bd_ref[...],
                          preferred_element_type=jnp.float32)


def _make_tc_a2_body(nr):
    def body(dp_ref, h1_ref, p16_ref, g1_ref, d32_ref, d16_ref):
        deg = dp_ref[0, :nr] + dp_ref[1, :nr] + 1.0
        d32 = lax.rsqrt(deg)
        d32_ref[...] = d32
        d16_ref[...] = jnp.dot(d32, p16_ref[...],
                               preferred_element_type=jnp.float32)
        g1_ref[...] = d32 * h1_ref[...]
    return body


def _make_tc_b_body(nr):
    def body(ap_ref, g1_ref, d32_ref, d16_ref, b1_ref, w2bd_ref, g2_ref):
        t = ap_ref[0, :nr] + ap_ref[1, :nr] + g1_ref[...]
        z1 = jnp.maximum(d32_ref[...] * t + b1_ref[...], 0.0)
        h2 = jnp.dot(z1, w2bd_ref[...], preferred_element_type=jnp.float32)
        g2_ref[...] = d16_ref[...] * h2
    return body


def _make_tc_c_body(nr, ncls):
    def body(ap_ref, g2_ref, d16_ref, b2_ref, wcbd_ref, bc_ref, gg_ref,
             out_ref):
        t = ap_ref[0, :nr] + ap_ref[1, :nr] + g2_ref[...]
        z2 = jnp.maximum(d16_ref[...] * t + b2_ref[...], 0.0)
        lp = jnp.dot(z2, wcbd_ref[...],
                     preferred_element_type=jnp.float32) + bc_ref[...]
        m = lp
        sh = 1
        while sh < 8 * ncls:
            m = jnp.maximum(m, pltpu.roll(m, sh, 1))
            sh *= 2
        ex = jnp.exp(lp - m)
        s = jnp.dot(ex, gg_ref[...], preferred_element_type=jnp.float32)
        out_ref[...] = lp - m - jnp.log(s)
    return body


def kernel(x, edge_index, W1, b1, W2, b2, Wc, bc):
    n, f = x.shape
    e = edge_index.shape[1]
    d1 = W1.shape[1]
    d2 = W2.shape[1]
    ncls = Wc.shape[1]

    n_chunks = e // _CH
    assert n_chunks * _CH == e
    n_pad = -(-n // (8 * _NS)) * (8 * _NS)

    ei = edge_index.astype(jnp.int32)
    src2d = ei[0].reshape(n_chunks, _CH)
    dst2d = ei[1].reshape(n_chunks, _CH)

    ones_dd = jnp.ones((_CH * _DD,), jnp.float32).reshape(_CH, _DD)
    z_d1 = jnp.zeros((n_pad * d1,), jnp.float32).reshape(n_pad, d1)
    z_d2 = jnp.zeros((n_pad * d2,), jnp.float32).reshape(n_pad, d2)
    z_dd = z_d1 if _DD == d1 else jnp.zeros(
        (n_pad * _DD,), jnp.float32).reshape(n_pad, _DD)

    i1 = jnp.arange(8 * d1)
    i2 = jnp.arange(8 * d2)
    ic = jnp.arange(8 * ncls)
    p16 = (i1[:, None] == (i2[None, :] // d2) * d1).astype(jnp.float32)
    gg = ((ic[:, None] // ncls) == (ic[None, :] // ncls)).astype(jnp.float32)

    eye8 = jnp.eye(8, dtype=jnp.float32)
    w1bd = jnp.kron(eye8, W1)
    w2bd = jnp.kron(eye8, W2)
    wcbd = jnp.kron(eye8, Wc)
    b1t = jnp.tile(b1, 8).reshape(1, 8 * d1)
    bct = jnp.tile(bc, 8).reshape(1, 8 * ncls)
    b2t = jnp.tile(b2, 8).reshape(1, 8 * d2)

    nr = n // 8
    nrp1 = n_pad * d1 // (8 * d1)
    nrp2 = n_pad * d2 // (8 * d2)
    nrd = n_pad * _DD // 128

    xv = x.reshape(nr, 8 * f)

    h1 = pl.pallas_call(
        _tc_a1_body,
        out_shape=jax.ShapeDtypeStruct((nr, 8 * d1), jnp.float32),
    )(xv, w1bd)

    deg_p = _make_deg_kernel(n_pad, n_chunks)(dst2d, ones_dd, z_dd)
    deg_pk = deg_p.reshape(_NC, n_pad * _DD // (8 * d1), 8 * d1)

    g1, d32, d16 = pl.pallas_call(
        _make_tc_a2_body(nr),
        out_shape=[
            jax.ShapeDtypeStruct((nr, 8 * d1), jnp.float32),
            jax.ShapeDtypeStruct((nr, 8 * d1), jnp.float32),
            jax.ShapeDtypeStruct((nr, 8 * d2), jnp.float32),
        ],
    )(deg_pk, h1, p16)

    agg1 = _make_agg_kernel(n_pad, n_chunks, d1)(
        g1.reshape(n, d1), src2d, dst2d, z_d1)
    agg1_pk = agg1.reshape(_NC, nrp1, 8 * d1)

    g2 = pl.pallas_call(
        _make_tc_b_body(nr),
        out_shape=jax.ShapeDtypeStruct((nr, 8 * d2), jnp.float32),
    )(agg1_pk, g1, d32, d16, b1t, w2bd)

    agg2 = _make_agg_kernel(n_pad, n_chunks, d2)(
        g2.reshape(n, d2), src2d, dst2d, z_d2)
    agg2_pk = agg2.reshape(_NC, nrp2, 8 * d2)

    out_pk = pl.pallas_call(
        _make_tc_c_body(nr, ncls),
        out_shape=jax.ShapeDtypeStruct((nr, 8 * ncls), jnp.float32),
    )(agg2_pk, g2, d16, b2t, wcbd, bct, gg)

    return out_pk.reshape(n, ncls)


# speedup vs baseline: 59.9647x; 1.0652x over previous
import jax
import jax.numpy as jnp
from jax import lax
from jax.experimental import pallas as pl
from jax.experimental.pallas import tpu as pltpu
from jax.experimental.pallas import tpu_sc as plsc

_NC = 2
_NS = 16
_CH = 128
_DD = 32


def _sc_mesh():
    return plsc.VectorSubcoreMesh(
        core_axis_name="c", subcore_axis_name="s",
        num_cores=_NC, num_subcores=_NS)


def _tile_chunks(n_chunks):
    base = n_chunks // (_NC * _NS)
    extra = n_chunks - base * _NC * _NS

    def start_cnt(c, s):
        t = c * _NS + s
        start = base * t + jnp.minimum(t, extra)
        cnt = base + jnp.where(t < extra, 1, 0)
        return start, cnt

    return base, extra, start_cnt


def _make_deg_kernel(n_pad, n_chunks):
    rows = n_pad // _NS
    base, extra, start_cnt = _tile_chunks(n_chunks)
    kmax = base + (1 if extra else 0)

    def body(dst_hbm, ones_hbm, zeros_hbm, out_hbm, dst_v, ones_v, ssems, acc):
        c = lax.axis_index("c")
        s = lax.axis_index("s")
        start, cnt = start_cnt(c, s)
        r0 = s * rows
        pltpu.sync_copy(zeros_hbm.at[pl.ds(r0, rows)], acc.at[pl.ds(r0, rows)])

        @pl.when(cnt == kmax)
        def _():
            pltpu.sync_copy(dst_hbm.at[pl.ds(start, kmax)], dst_v)

        @pl.when(cnt == base)
        def _():
            pltpu.sync_copy(dst_hbm.at[pl.ds(start, base)],
                            dst_v.at[pl.ds(0, base)])

        pltpu.sync_copy(ones_hbm, ones_v)
        plsc.subcore_barrier()

        def step(j, carry):
            @pl.when(j >= 2)
            def _():
                jm = j - 2
                pltpu.make_async_copy(
                    ones_v, acc.at[dst_v.at[jm]], ssems.at[lax.rem(jm, 2)]
                ).wait()
            pltpu.async_copy(ones_v, acc.at[dst_v.at[j]],
                             ssems.at[lax.rem(j, 2)], add=True)
            return carry

        lax.fori_loop(0, cnt, step, 0)
        for dj in (cnt - 2, cnt - 1):
            pltpu.make_async_copy(
                ones_v, acc.at[dst_v.at[dj]], ssems.at[lax.rem(dj, 2)]).wait()
        plsc.subcore_barrier()
        pltpu.sync_copy(acc.at[pl.ds(r0, rows)],
                        out_hbm.at[c, pl.ds(r0, rows)])

    return pl.kernel(
        body,
        out_type=jax.ShapeDtypeStruct((_NC, n_pad, _DD), jnp.float32),
        mesh=_sc_mesh(),
        compiler_params=pltpu.CompilerParams(use_tc_tiling_on_sc=False),
        scratch_types=[
            pltpu.VMEM((kmax, _CH), jnp.int32),
            pltpu.VMEM((_CH, _DD), jnp.float32),
            pltpu.SemaphoreType.DMA((2,)),
            pltpu.VMEM_SHARED((n_pad, _DD), jnp.float32),
        ],
    )


def _make_agg_kernel(n, n_pad, n_chunks, d_feat):
    rows = n_pad // _NS
    base, extra, start_cnt = _tile_chunks(n_chunks)
    kmax = base + (1 if extra else 0)

    def body(g_hbm, src_hbm, dst_hbm, zeros_hbm, out_hbm,
             src_v, dst_v, rows_v, gsems, ssems, acc):
        c = lax.axis_index("c")
        s = lax.axis_index("s")
        start, cnt = start_cnt(c, s)
        r0 = s * rows
        pltpu.sync_copy(zeros_hbm.at[pl.ds(r0, rows)], acc.at[pl.ds(r0, rows)])

        @pl.when(cnt == kmax)
        def _():
            pltpu.sync_copy(src_hbm.at[pl.ds(start, kmax)], src_v)
            pltpu.sync_copy(dst_hbm.at[pl.ds(start, kmax)], dst_v)

        @pl.when(cnt == base)
        def _():
            pltpu.sync_copy(src_hbm.at[pl.ds(start, base)],
                            src_v.at[pl.ds(0, base)])
            pltpu.sync_copy(dst_hbm.at[pl.ds(start, base)],
                            dst_v.at[pl.ds(0, base)])

        plsc.subcore_barrier()

        pltpu.async_copy(g_hbm.at[src_v.at[0]], rows_v.at[0], gsems.at[0])
        pltpu.async_copy(g_hbm.at[src_v.at[1]], rows_v.at[1], gsems.at[1])

        def step(j, carry):
            b4 = lax.rem(j, 4)
            pltpu.make_async_copy(
                g_hbm.at[src_v.at[j]], rows_v.at[b4], gsems.at[b4]).wait()

            @pl.when(j >= 2)
            def _():
                jm = j - 2
                pltpu.make_async_copy(
                    rows_v.at[lax.rem(jm, 4)], acc.at[dst_v.at[jm]],
                    ssems.at[lax.rem(jm, 2)]).wait()

            @pl.when(j + 2 < cnt)
            def _():
                jp = j + 2
                bp = lax.rem(jp, 4)
                pltpu.async_copy(
                    g_hbm.at[src_v.at[jp]], rows_v.at[bp], gsems.at[bp])

            pltpu.async_copy(rows_v.at[b4], acc.at[dst_v.at[j]],
                             ssems.at[lax.rem(j, 2)], add=True)
            return carry

        lax.fori_loop(0, cnt, step, 0)
        for dj in (cnt - 2, cnt - 1):
            pltpu.make_async_copy(
                rows_v.at[lax.rem(dj, 4)], acc.at[dst_v.at[dj]],
                ssems.at[lax.rem(dj, 2)]).wait()
        plsc.subcore_barrier()
        pltpu.sync_copy(acc.at[pl.ds(r0, rows)],
                        out_hbm.at[c, pl.ds(r0, rows)])

    return pl.kernel(
        body,
        out_type=jax.ShapeDtypeStruct((_NC, n_pad, d_feat), jnp.float32),
        mesh=_sc_mesh(),
        compiler_params=pltpu.CompilerParams(use_tc_tiling_on_sc=False),
        scratch_types=[
            pltpu.VMEM((kmax, _CH), jnp.int32),
            pltpu.VMEM((kmax, _CH), jnp.int32),
            pltpu.VMEM((4, _CH, d_feat), jnp.float32),
            pltpu.SemaphoreType.DMA((4,)),
            pltpu.SemaphoreType.DMA((2,)),
            pltpu.VMEM_SHARED((n_pad, d_feat), jnp.float32),
        ],
    )


def _make_edge_repack_body(n_chunks):
    def body(ei_ref, src_ref, dst_ref):
        src_ref[...] = jnp.reshape(ei_ref[0], (n_chunks, _CH))
        dst_ref[...] = jnp.reshape(ei_ref[1], (n_chunks, _CH))
    return body


def _tc_a1_body(xv_ref, w1bd_ref, h1_ref):
    h1_ref[...] = jnp.dot(xv_ref[...], w1bd_ref[...],
                          preferred_element_type=jnp.float32)


def _make_tc_a2_body(nr):
    def body(dp_ref, h1_ref, p16_ref, g1_ref, d32_ref, d16_ref):
        deg = dp_ref[0, :nr] + dp_ref[1, :nr] + 1.0
        d32 = lax.rsqrt(deg)
        d32_ref[...] = d32
        d16_ref[...] = jnp.dot(d32, p16_ref[...],
                               preferred_element_type=jnp.float32)
        g1_ref[...] = d32 * h1_ref[...]
    return body


def _make_tc_b_body(nr):
    def body(ap_ref, g1_ref, d32_ref, d16_ref, b1_ref, w2bd_ref, g2_ref):
        t = ap_ref[0, :nr] + ap_ref[1, :nr] + g1_ref[...]
        z1 = jnp.maximum(d32_ref[...] * t + b1_ref[...], 0.0)
        h2 = jnp.dot(z1, w2bd_ref[...], preferred_element_type=jnp.float32)
        g2_ref[...] = d16_ref[...] * h2
    return body


def _make_tc_c_body(nr, ncls):
    def body(ap_ref, g2_ref, d16_ref, b2_ref, wcbd_ref, bc_ref, gg_ref,
             out_ref):
        t = ap_ref[0, :nr] + ap_ref[1, :nr] + g2_ref[...]
        z2 = jnp.maximum(d16_ref[...] * t + b2_ref[...], 0.0)
        lp = jnp.dot(z2, wcbd_ref[...],
                     preferred_element_type=jnp.float32) + bc_ref[...]
        m = lp
        sh = 1
        while sh < 8 * ncls:
            m = jnp.maximum(m, pltpu.roll(m, sh, 1))
            sh *= 2
        ex = jnp.exp(lp - m)
        s = jnp.dot(ex, gg_ref[...], preferred_element_type=jnp.float32)
        out_ref[...] = lp - m - jnp.log(s)
    return body


def kernel(x, edge_index, W1, b1, W2, b2, Wc, bc):
    n, f = x.shape
    e = edge_index.shape[1]
    d1 = W1.shape[1]
    d2 = W2.shape[1]
    ncls = Wc.shape[1]

    n_chunks = e // _CH
    assert n_chunks * _CH == e
    n_pad = -(-n // (8 * _NS)) * (8 * _NS)

    ei = edge_index.astype(jnp.int32)
    src2d, dst2d = pl.pallas_call(
        _make_edge_repack_body(n_chunks),
        out_shape=[
            jax.ShapeDtypeStruct((n_chunks, _CH), jnp.int32),
            jax.ShapeDtypeStruct((n_chunks, _CH), jnp.int32),
        ],
    )(ei)

    ones_dd = jnp.ones((_CH * _DD,), jnp.float32).reshape(_CH, _DD)
    z_d1 = jnp.zeros((n_pad * d1,), jnp.float32).reshape(n_pad, d1)
    z_d2 = jnp.zeros((n_pad * d2,), jnp.float32).reshape(n_pad, d2)
    z_dd = z_d1 if _DD == d1 else jnp.zeros(
        (n_pad * _DD,), jnp.float32).reshape(n_pad, _DD)

    i1 = jnp.arange(8 * d1)
    i2 = jnp.arange(8 * d2)
    ic = jnp.arange(8 * ncls)
    p16 = (i1[:, None] == (i2[None, :] // d2) * d1).astype(jnp.float32)
    gg = ((ic[:, None] // ncls) == (ic[None, :] // ncls)).astype(jnp.float32)

    eye8 = jnp.eye(8, dtype=jnp.float32)
    w1bd = jnp.kron(eye8, W1)
    w2bd = jnp.kron(eye8, W2)
    wcbd = jnp.kron(eye8, Wc)
    b1t = jnp.tile(b1, 8).reshape(1, 8 * d1)
    bct = jnp.tile(bc, 8).reshape(1, 8 * ncls)
    b2t = jnp.tile(b2, 8).reshape(1, 8 * d2)

    nr = n // 8
    nrp1 = n_pad * d1 // (8 * d1)
    nrp2 = n_pad * d2 // (8 * d2)
    nrd = n_pad * _DD // 128

    xv = x.reshape(nr, 8 * f)

    h1 = pl.pallas_call(
        _tc_a1_body,
        out_shape=jax.ShapeDtypeStruct((nr, 8 * d1), jnp.float32),
    )(xv, w1bd)

    deg_p = _make_deg_kernel(n_pad, n_chunks)(dst2d, ones_dd, z_dd)
    deg_pk = deg_p.reshape(_NC, n_pad * _DD // (8 * d1), 8 * d1)

    g1, d32, d16 = pl.pallas_call(
        _make_tc_a2_body(nr),
        out_shape=[
            jax.ShapeDtypeStruct((nr, 8 * d1), jnp.float32),
            jax.ShapeDtypeStruct((nr, 8 * d1), jnp.float32),
            jax.ShapeDtypeStruct((nr, 8 * d2), jnp.float32),
        ],
    )(deg_pk, h1, p16)

    agg1 = _make_agg_kernel(n, n_pad, n_chunks, d1)(
        g1.reshape(n, d1), src2d, dst2d, z_d1)
    agg1_pk = agg1.reshape(_NC, nrp1, 8 * d1)

    g2 = pl.pallas_call(
        _make_tc_b_body(nr),
        out_shape=jax.ShapeDtypeStruct((nr, 8 * d2), jnp.float32),
    )(agg1_pk, g1, d32, d16, b1t, w2bd)

    agg2 = _make_agg_kernel(n, n_pad, n_chunks, d2)(
        g2.reshape(n, d2), src2d, dst2d, z_d2)
    agg2_pk = agg2.reshape(_NC, nrp2, 8 * d2)

    out_pk = pl.pallas_call(
        _make_tc_c_body(nr, ncls),
        out_shape=jax.ShapeDtypeStruct((nr, 8 * ncls), jnp.float32),
    )(agg2_pk, g2, d16, b2t, wcbd, bct, gg)

    return out_pk.reshape(n, ncls)


# speedup vs baseline: 70.4576x; 1.1750x over previous
import jax
import jax.numpy as jnp
from jax import lax
from jax.experimental import pallas as pl
from jax.experimental.pallas import tpu as pltpu
from jax.experimental.pallas import tpu_sc as plsc

_NC = 2
_NS = 16
_CH = 128
_DD = 32


def _sc_mesh():
    return plsc.VectorSubcoreMesh(
        core_axis_name="c", subcore_axis_name="s",
        num_cores=_NC, num_subcores=_NS)


def _tile_chunks(n_chunks):
    base = n_chunks // (_NC * _NS)
    extra = n_chunks - base * _NC * _NS

    def start_cnt(c, s):
        t = c * _NS + s
        start = base * t + jnp.minimum(t, extra)
        cnt = base + jnp.where(t < extra, 1, 0)
        return start, cnt

    return base, extra, start_cnt


def _make_deg_kernel(n_pad, n_chunks):
    rows = n_pad // _NS
    base, extra, start_cnt = _tile_chunks(n_chunks)
    kmax = base + (1 if extra else 0)

    def body(dst_hbm, ones_hbm, zeros_hbm, out_hbm, dst_v, ones_v, ssems, acc):
        c = lax.axis_index("c")
        s = lax.axis_index("s")
        start, cnt = start_cnt(c, s)
        r0 = s * rows
        pltpu.sync_copy(zeros_hbm.at[pl.ds(r0, rows)], acc.at[pl.ds(r0, rows)])

        @pl.when(cnt == kmax)
        def _():
            pltpu.sync_copy(dst_hbm.at[pl.ds(start, kmax)], dst_v)

        @pl.when(cnt == base)
        def _():
            pltpu.sync_copy(dst_hbm.at[pl.ds(start, base)],
                            dst_v.at[pl.ds(0, base)])

        pltpu.sync_copy(ones_hbm, ones_v)
        plsc.subcore_barrier()

        def step(j, carry):
            @pl.when(j >= 4)
            def _():
                jm = j - 4
                pltpu.make_async_copy(
                    ones_v, acc.at[dst_v.at[jm]], ssems.at[lax.rem(jm, 4)]
                ).wait()
            pltpu.async_copy(ones_v, acc.at[dst_v.at[j]],
                             ssems.at[lax.rem(j, 4)], add=True)
            return carry

        lax.fori_loop(0, cnt, step, 0)
        for dj in (cnt - 4, cnt - 3, cnt - 2, cnt - 1):
            pltpu.make_async_copy(
                ones_v, acc.at[dst_v.at[dj]], ssems.at[lax.rem(dj, 4)]).wait()
        plsc.subcore_barrier()
        pltpu.sync_copy(acc.at[pl.ds(r0, rows)],
                        out_hbm.at[c, pl.ds(r0, rows)])

    return pl.kernel(
        body,
        out_type=jax.ShapeDtypeStruct((_NC, n_pad, _DD), jnp.float32),
        mesh=_sc_mesh(),
        compiler_params=pltpu.CompilerParams(use_tc_tiling_on_sc=False),
        scratch_types=[
            pltpu.VMEM((kmax, _CH), jnp.int32),
            pltpu.VMEM((_CH, _DD), jnp.float32),
            pltpu.SemaphoreType.DMA((4,)),
            pltpu.VMEM_SHARED((n_pad, _DD), jnp.float32),
        ],
    )


def _make_agg_kernel(n, n_pad, n_chunks, d_feat):
    rows = n_pad // _NS
    base, extra, start_cnt = _tile_chunks(n_chunks)
    kmax = base + (1 if extra else 0)

    def body(g_hbm, src_hbm, dst_hbm, zeros_hbm, out_hbm,
             src_v, dst_v, rows_v, gsems, ssems, acc):
        c = lax.axis_index("c")
        s = lax.axis_index("s")
        start, cnt = start_cnt(c, s)
        r0 = s * rows
        pltpu.sync_copy(zeros_hbm.at[pl.ds(r0, rows)], acc.at[pl.ds(r0, rows)])

        @pl.when(cnt == kmax)
        def _():
            pltpu.sync_copy(src_hbm.at[pl.ds(start, kmax)], src_v)
            pltpu.sync_copy(dst_hbm.at[pl.ds(start, kmax)], dst_v)

        @pl.when(cnt == base)
        def _():
            pltpu.sync_copy(src_hbm.at[pl.ds(start, base)],
                            src_v.at[pl.ds(0, base)])
            pltpu.sync_copy(dst_hbm.at[pl.ds(start, base)],
                            dst_v.at[pl.ds(0, base)])

        plsc.subcore_barrier()

        for p in range(4):
            pltpu.async_copy(g_hbm.at[src_v.at[p]], rows_v.at[p], gsems.at[p])

        def step(j, carry):
            b8 = lax.rem(j, 8)
            pltpu.make_async_copy(
                g_hbm.at[src_v.at[j]], rows_v.at[b8], gsems.at[b8]).wait()

            @pl.when(j >= 4)
            def _():
                jm = j - 4
                pltpu.make_async_copy(
                    rows_v.at[lax.rem(jm, 8)], acc.at[dst_v.at[jm]],
                    ssems.at[lax.rem(jm, 4)]).wait()

            @pl.when(j + 4 < cnt)
            def _():
                jp = j + 4
                bp = lax.rem(jp, 8)
                pltpu.async_copy(
                    g_hbm.at[src_v.at[jp]], rows_v.at[bp], gsems.at[bp])

            pltpu.async_copy(rows_v.at[b8], acc.at[dst_v.at[j]],
                             ssems.at[lax.rem(j, 4)], add=True)
            return carry

        lax.fori_loop(0, cnt, step, 0)
        for dj in (cnt - 4, cnt - 3, cnt - 2, cnt - 1):
            pltpu.make_async_copy(
                rows_v.at[lax.rem(dj, 8)], acc.at[dst_v.at[dj]],
                ssems.at[lax.rem(dj, 4)]).wait()
        plsc.subcore_barrier()
        pltpu.sync_copy(acc.at[pl.ds(r0, rows)],
                        out_hbm.at[c, pl.ds(r0, rows)])

    return pl.kernel(
        body,
        out_type=jax.ShapeDtypeStruct((_NC, n_pad, d_feat), jnp.float32),
        mesh=_sc_mesh(),
        compiler_params=pltpu.CompilerParams(use_tc_tiling_on_sc=False),
        scratch_types=[
            pltpu.VMEM((kmax, _CH), jnp.int32),
            pltpu.VMEM((kmax, _CH), jnp.int32),
            pltpu.VMEM((8, _CH, d_feat), jnp.float32),
            pltpu.SemaphoreType.DMA((8,)),
            pltpu.SemaphoreType.DMA((4,)),
            pltpu.VMEM_SHARED((n_pad, d_feat), jnp.float32),
        ],
    )


def _make_edge_repack_body(n_chunks):
    def body(ei_ref, src_ref, dst_ref):
        src_ref[...] = jnp.reshape(ei_ref[0], (n_chunks, _CH))
        dst_ref[...] = jnp.reshape(ei_ref[1], (n_chunks, _CH))
    return body


def _tc_a1_body(xv_ref, w1bd_ref, h1_ref):
    h1_ref[...] = jnp.dot(xv_ref[...], w1bd_ref[...],
                          preferred_element_type=jnp.float32)


def _make_tc_a2_body(nr):
    def body(dp_ref, h1_ref, p16_ref, g1_ref, d32_ref, d16_ref):
        deg = dp_ref[0, :nr] + dp_ref[1, :nr] + 1.0
        d32 = lax.rsqrt(deg)
        d32_ref[...] = d32
        d16_ref[...] = jnp.dot(d32, p16_ref[...],
                               preferred_element_type=jnp.float32)
        g1_ref[...] = d32 * h1_ref[...]
    return body


def _make_tc_b_body(nr):
    def body(ap_ref, g1_ref, d32_ref, d16_ref, b1_ref, w2bd_ref, g2_ref):
        t = ap_ref[0, :nr] + ap_ref[1, :nr] + g1_ref[...]
        z1 = jnp.maximum(d32_ref[...] * t + b1_ref[...], 0.0)
        h2 = jnp.dot(z1, w2bd_ref[...], preferred_element_type=jnp.float32)
        g2_ref[...] = d16_ref[...] * h2
    return body


def _make_tc_c_body(nr, ncls):
    def body(ap_ref, g2_ref, d16_ref, b2_ref, wcbd_ref, bc_ref, gg_ref,
             out_ref):
        t = ap_ref[0, :nr] + ap_ref[1, :nr] + g2_ref[...]
        z2 = jnp.maximum(d16_ref[...] * t + b2_ref[...], 0.0)
        lp = jnp.dot(z2, wcbd_ref[...],
                     preferred_element_type=jnp.float32) + bc_ref[...]
        m = lp
        sh = 1
        while sh < 8 * ncls:
            m = jnp.maximum(m, pltpu.roll(m, sh, 1))
            sh *= 2
        ex = jnp.exp(lp - m)
        s = jnp.dot(ex, gg_ref[...], preferred_element_type=jnp.float32)
        out_ref[...] = lp - m - jnp.log(s)
    return body


def kernel(x, edge_index, W1, b1, W2, b2, Wc, bc):
    n, f = x.shape
    e = edge_index.shape[1]
    d1 = W1.shape[1]
    d2 = W2.shape[1]
    ncls = Wc.shape[1]

    n_chunks = e // _CH
    assert n_chunks * _CH == e
    n_pad = -(-n // (8 * _NS)) * (8 * _NS)

    ei = edge_index.astype(jnp.int32)
    src2d, dst2d = pl.pallas_call(
        _make_edge_repack_body(n_chunks),
        out_shape=[
            jax.ShapeDtypeStruct((n_chunks, _CH), jnp.int32),
            jax.ShapeDtypeStruct((n_chunks, _CH), jnp.int32),
        ],
    )(ei)

    ones_dd = jnp.ones((_CH * _DD,), jnp.float32).reshape(_CH, _DD)
    z_d1 = jnp.zeros((n_pad * d1,), jnp.float32).reshape(n_pad, d1)
    z_d2 = jnp.zeros((n_pad * d2,), jnp.float32).reshape(n_pad, d2)
    z_dd = z_d1 if _DD == d1 else jnp.zeros(
        (n_pad * _DD,), jnp.float32).reshape(n_pad, _DD)

    i1 = jnp.arange(8 * d1)
    i2 = jnp.arange(8 * d2)
    ic = jnp.arange(8 * ncls)
    p16 = (i1[:, None] == (i2[None, :] // d2) * d1).astype(jnp.float32)
    gg = ((ic[:, None] // ncls) == (ic[None, :] // ncls)).astype(jnp.float32)

    eye8 = jnp.eye(8, dtype=jnp.float32)
    w1bd = jnp.kron(eye8, W1)
    w2bd = jnp.kron(eye8, W2)
    wcbd = jnp.kron(eye8, Wc)
    b1t = jnp.tile(b1, 8).reshape(1, 8 * d1)
    bct = jnp.tile(bc, 8).reshape(1, 8 * ncls)
    b2t = jnp.tile(b2, 8).reshape(1, 8 * d2)

    nr = n // 8
    nrp1 = n_pad * d1 // (8 * d1)
    nrp2 = n_pad * d2 // (8 * d2)
    nrd = n_pad * _DD // 128

    xv = x.reshape(nr, 8 * f)

    h1 = pl.pallas_call(
        _tc_a1_body,
        out_shape=jax.ShapeDtypeStruct((nr, 8 * d1), jnp.float32),
    )(xv, w1bd)

    deg_p = _make_deg_kernel(n_pad, n_chunks)(dst2d, ones_dd, z_dd)
    deg_pk = deg_p.reshape(_NC, n_pad * _DD // (8 * d1), 8 * d1)

    g1, d32, d16 = pl.pallas_call(
        _make_tc_a2_body(nr),
        out_shape=[
            jax.ShapeDtypeStruct((nr, 8 * d1), jnp.float32),
            jax.ShapeDtypeStruct((nr, 8 * d1), jnp.float32),
            jax.ShapeDtypeStruct((nr, 8 * d2), jnp.float32),
        ],
    )(deg_pk, h1, p16)

    agg1 = _make_agg_kernel(n, n_pad, n_chunks, d1)(
        g1.reshape(n, d1), src2d, dst2d, z_d1)
    agg1_pk = agg1.reshape(_NC, nrp1, 8 * d1)

    g2 = pl.pallas_call(
        _make_tc_b_body(nr),
        out_shape=jax.ShapeDtypeStruct((nr, 8 * d2), jnp.float32),
    )(agg1_pk, g1, d32, d16, b1t, w2bd)

    agg2 = _make_agg_kernel(n, n_pad, n_chunks, d2)(
        g2.reshape(n, d2), src2d, dst2d, z_d2)
    agg2_pk = agg2.reshape(_NC, nrp2, 8 * d2)

    out_pk = pl.pallas_call(
        _make_tc_c_body(nr, ncls),
        out_shape=jax.ShapeDtypeStruct((nr, 8 * ncls), jnp.float32),
    )(agg2_pk, g2, d16, b2t, wcbd, bct, gg)

    return out_pk.reshape(n, ncls)


# speedup vs baseline: 74.5048x; 1.0574x over previous
import jax
import jax.numpy as jnp
from jax import lax
from jax.experimental import pallas as pl
from jax.experimental.pallas import tpu as pltpu
from jax.experimental.pallas import tpu_sc as plsc

_NC = 2
_NS = 16
_CH = 128
_DD = 32


def _sc_mesh():
    return plsc.VectorSubcoreMesh(
        core_axis_name="c", subcore_axis_name="s",
        num_cores=_NC, num_subcores=_NS)


def _tile_chunks(n_chunks):
    base = n_chunks // (_NC * _NS)
    extra = n_chunks - base * _NC * _NS

    def start_cnt(c, s):
        t = c * _NS + s
        start = base * t + jnp.minimum(t, extra)
        cnt = base + jnp.where(t < extra, 1, 0)
        return start, cnt

    return base, extra, start_cnt


def _make_deg_kernel(n_pad, n_chunks):
    rows = n_pad // _NS
    base, extra, start_cnt = _tile_chunks(n_chunks)
    kmax = base + (1 if extra else 0)

    def body(dst_hbm, ones_hbm, zeros_hbm, out_hbm, dst_v, ones_v, ssems, acc):
        c = lax.axis_index("c")
        s = lax.axis_index("s")
        start, cnt = start_cnt(c, s)
        r0 = s * rows
        pltpu.sync_copy(zeros_hbm.at[pl.ds(r0, rows)], acc.at[pl.ds(r0, rows)])

        @pl.when(cnt == kmax)
        def _():
            pltpu.sync_copy(dst_hbm.at[pl.ds(start, kmax)], dst_v)

        @pl.when(cnt == base)
        def _():
            pltpu.sync_copy(dst_hbm.at[pl.ds(start, base)],
                            dst_v.at[pl.ds(0, base)])

        pltpu.sync_copy(ones_hbm, ones_v)
        plsc.subcore_barrier()

        def step(j, carry):
            @pl.when(j >= 4)
            def _():
                jm = j - 4
                pltpu.make_async_copy(
                    ones_v, acc.at[dst_v.at[jm]], ssems.at[lax.rem(jm, 4)]
                ).wait()
            pltpu.async_copy(ones_v, acc.at[dst_v.at[j]],
                             ssems.at[lax.rem(j, 4)], add=True)
            return carry

        lax.fori_loop(0, cnt, step, 0)
        for dj in (cnt - 4, cnt - 3, cnt - 2, cnt - 1):
            pltpu.make_async_copy(
                ones_v, acc.at[dst_v.at[dj]], ssems.at[lax.rem(dj, 4)]).wait()
        plsc.subcore_barrier()
        pltpu.sync_copy(acc.at[pl.ds(r0, rows)],
                        out_hbm.at[c, pl.ds(r0, rows)])

    return pl.kernel(
        body,
        out_type=jax.ShapeDtypeStruct((_NC, n_pad, _DD), jnp.float32),
        mesh=_sc_mesh(),
        compiler_params=pltpu.CompilerParams(use_tc_tiling_on_sc=False),
        scratch_types=[
            pltpu.VMEM((kmax, _CH), jnp.int32),
            pltpu.VMEM((_CH, _DD), jnp.float32),
            pltpu.SemaphoreType.DMA((4,)),
            pltpu.VMEM_SHARED((n_pad, _DD), jnp.float32),
        ],
    )


def _make_agg_kernel(n, n_pad, n_chunks, d_feat):
    rows = n_pad // _NS
    base, extra, start_cnt = _tile_chunks(n_chunks)
    kmax = base + (1 if extra else 0)

    def body(g_hbm, src_hbm, dst_hbm, zeros_hbm, out_hbm,
             src_v, dst_v, rows_v, gsems, ssems, acc):
        c = lax.axis_index("c")
        s = lax.axis_index("s")
        start, cnt = start_cnt(c, s)
        r0 = s * rows
        pltpu.sync_copy(zeros_hbm.at[pl.ds(r0, rows)], acc.at[pl.ds(r0, rows)])

        @pl.when(cnt == kmax)
        def _():
            pltpu.sync_copy(src_hbm.at[pl.ds(start, kmax)], src_v)
            pltpu.sync_copy(dst_hbm.at[pl.ds(start, kmax)], dst_v)

        @pl.when(cnt == base)
        def _():
            pltpu.sync_copy(src_hbm.at[pl.ds(start, base)],
                            src_v.at[pl.ds(0, base)])
            pltpu.sync_copy(dst_hbm.at[pl.ds(start, base)],
                            dst_v.at[pl.ds(0, base)])

        plsc.subcore_barrier()

        for p in range(8):
            pltpu.async_copy(g_hbm.at[src_v.at[p]], rows_v.at[p], gsems.at[p])

        def step(j, carry):
            b16 = lax.rem(j, 16)
            pltpu.make_async_copy(
                g_hbm.at[src_v.at[j]], rows_v.at[b16], gsems.at[b16]).wait()

            @pl.when(j >= 8)
            def _():
                jm = j - 8
                pltpu.make_async_copy(
                    rows_v.at[lax.rem(jm, 16)], acc.at[dst_v.at[jm]],
                    ssems.at[lax.rem(jm, 8)]).wait()

            @pl.when(j + 8 < cnt)
            def _():
                jp = j + 8
                bp = lax.rem(jp, 16)
                pltpu.async_copy(
                    g_hbm.at[src_v.at[jp]], rows_v.at[bp], gsems.at[bp])

            pltpu.async_copy(rows_v.at[b16], acc.at[dst_v.at[j]],
                             ssems.at[lax.rem(j, 8)], add=True)
            return carry

        lax.fori_loop(0, cnt, step, 0)
        for dj in range(8):
            pltpu.make_async_copy(
                rows_v.at[lax.rem(cnt - 8 + dj, 16)],
                acc.at[dst_v.at[cnt - 8 + dj]],
                ssems.at[lax.rem(cnt - 8 + dj, 8)]).wait()
        plsc.subcore_barrier()
        pltpu.sync_copy(acc.at[pl.ds(r0, rows)],
                        out_hbm.at[c, pl.ds(r0, rows)])

    return pl.kernel(
        body,
        out_type=jax.ShapeDtypeStruct((_NC, n_pad, d_feat), jnp.float32),
        mesh=_sc_mesh(),
        compiler_params=pltpu.CompilerParams(use_tc_tiling_on_sc=False),
        scratch_types=[
            pltpu.VMEM((kmax, _CH), jnp.int32),
            pltpu.VMEM((kmax, _CH), jnp.int32),
            pltpu.VMEM((16, _CH, d_feat), jnp.float32),
            pltpu.SemaphoreType.DMA((16,)),
            pltpu.SemaphoreType.DMA((8,)),
            pltpu.VMEM_SHARED((n_pad, d_feat), jnp.float32),
        ],
    )


def _make_edge_repack_body(n_chunks):
    def body(ei_ref, src_ref, dst_ref):
        src_ref[...] = jnp.reshape(ei_ref[0], (n_chunks, _CH))
        dst_ref[...] = jnp.reshape(ei_ref[1], (n_chunks, _CH))
    return body


def _tc_a1_body(xv_ref, w1bd_ref, h1_ref):
    h1_ref[...] = jnp.dot(xv_ref[...], w1bd_ref[...],
                          preferred_element_type=jnp.float32)


def _make_tc_a2_body(nr):
    def body(dp_ref, h1_ref, p16_ref, g1_ref, d32_ref, d16_ref):
        deg = dp_ref[0, :nr] + dp_ref[1, :nr] + 1.0
        d32 = lax.rsqrt(deg)
        d32_ref[...] = d32
        d16_ref[...] = jnp.dot(d32, p16_ref[...],
                               preferred_element_type=jnp.float32)
        g1_ref[...] = d32 * h1_ref[...]
    return body


def _make_tc_b_body(nr):
    def body(ap_ref, g1_ref, d32_ref, d16_ref, b1_ref, w2bd_ref, g2_ref):
        t = ap_ref[0, :nr] + ap_ref[1, :nr] + g1_ref[...]
        z1 = jnp.maximum(d32_ref[...] * t + b1_ref[...], 0.0)
        h2 = jnp.dot(z1, w2bd_ref[...], preferred_element_type=jnp.float32)
        g2_ref[...] = d16_ref[...] * h2
    return body


def _make_tc_c_body(nr, ncls):
    def body(ap_ref, g2_ref, d16_ref, b2_ref, wcbd_ref, bc_ref, gg_ref,
             out_ref):
        t = ap_ref[0, :nr] + ap_ref[1, :nr] + g2_ref[...]
        z2 = jnp.maximum(d16_ref[...] * t + b2_ref[...], 0.0)
        lp = jnp.dot(z2, wcbd_ref[...],
                     preferred_element_type=jnp.float32) + bc_ref[...]
        m = lp
        sh = 1
        while sh < 8 * ncls:
            m = jnp.maximum(m, pltpu.roll(m, sh, 1))
            sh *= 2
        ex = jnp.exp(lp - m)
        s = jnp.dot(ex, gg_ref[...], preferred_element_type=jnp.float32)
        out_ref[...] = lp - m - jnp.log(s)
    return body


def kernel(x, edge_index, W1, b1, W2, b2, Wc, bc):
    n, f = x.shape
    e = edge_index.shape[1]
    d1 = W1.shape[1]
    d2 = W2.shape[1]
    ncls = Wc.shape[1]

    n_chunks = e // _CH
    assert n_chunks * _CH == e
    n_pad = -(-n // (8 * _NS)) * (8 * _NS)

    ei = edge_index.astype(jnp.int32)
    src2d, dst2d = pl.pallas_call(
        _make_edge_repack_body(n_chunks),
        out_shape=[
            jax.ShapeDtypeStruct((n_chunks, _CH), jnp.int32),
            jax.ShapeDtypeStruct((n_chunks, _CH), jnp.int32),
        ],
    )(ei)

    ones_dd = jnp.ones((_CH * _DD,), jnp.float32).reshape(_CH, _DD)
    z_d1 = jnp.zeros((n_pad * d1,), jnp.float32).reshape(n_pad, d1)
    z_d2 = jnp.zeros((n_pad * d2,), jnp.float32).reshape(n_pad, d2)
    z_dd = z_d1 if _DD == d1 else jnp.zeros(
        (n_pad * _DD,), jnp.float32).reshape(n_pad, _DD)

    i1 = jnp.arange(8 * d1)
    i2 = jnp.arange(8 * d2)
    ic = jnp.arange(8 * ncls)
    p16 = (i1[:, None] == (i2[None, :] // d2) * d1).astype(jnp.float32)
    gg = ((ic[:, None] // ncls) == (ic[None, :] // ncls)).astype(jnp.float32)

    eye8 = jnp.eye(8, dtype=jnp.float32)
    w1bd = jnp.kron(eye8, W1)
    w2bd = jnp.kron(eye8, W2)
    wcbd = jnp.kron(eye8, Wc)
    b1t = jnp.tile(b1, 8).reshape(1, 8 * d1)
    bct = jnp.tile(bc, 8).reshape(1, 8 * ncls)
    b2t = jnp.tile(b2, 8).reshape(1, 8 * d2)

    nr = n // 8
    nrp1 = n_pad * d1 // (8 * d1)
    nrp2 = n_pad * d2 // (8 * d2)
    nrd = n_pad * _DD // 128

    xv = x.reshape(nr, 8 * f)

    h1 = pl.pallas_call(
        _tc_a1_body,
        out_shape=jax.ShapeDtypeStruct((nr, 8 * d1), jnp.float32),
    )(xv, w1bd)

    deg_p = _make_deg_kernel(n_pad, n_chunks)(dst2d, ones_dd, z_dd)
    deg_pk = deg_p.reshape(_NC, n_pad * _DD // (8 * d1), 8 * d1)

    g1, d32, d16 = pl.pallas_call(
        _make_tc_a2_body(nr),
        out_shape=[
            jax.ShapeDtypeStruct((nr, 8 * d1), jnp.float32),
            jax.ShapeDtypeStruct((nr, 8 * d1), jnp.float32),
            jax.ShapeDtypeStruct((nr, 8 * d2), jnp.float32),
        ],
    )(deg_pk, h1, p16)

    agg1 = _make_agg_kernel(n, n_pad, n_chunks, d1)(
        g1.reshape(n, d1), src2d, dst2d, z_d1)
    agg1_pk = agg1.reshape(_NC, nrp1, 8 * d1)

    g2 = pl.pallas_call(
        _make_tc_b_body(nr),
        out_shape=jax.ShapeDtypeStruct((nr, 8 * d2), jnp.float32),
    )(agg1_pk, g1, d32, d16, b1t, w2bd)

    agg2 = _make_agg_kernel(n, n_pad, n_chunks, d2)(
        g2.reshape(n, d2), src2d, dst2d, z_d2)
    agg2_pk = agg2.reshape(_NC, nrp2, 8 * d2)

    out_pk = pl.pallas_call(
        _make_tc_c_body(nr, ncls),
        out_shape=jax.ShapeDtypeStruct((nr, 8 * ncls), jnp.float32),
    )(agg2_pk, g2, d16, b2t, wcbd, bct, gg)

    return out_pk.reshape(n, ncls)
